# baseline (speedup 1.0000x reference)
import functools

import jax
import jax.numpy as jnp
from jax import lax
from jax.experimental import pallas as pl
from jax.experimental.pallas import tpu as pltpu
from jax.experimental.pallas import tpu_sc as plsc


def _make_sc_kernel(B, DZ, DS, V, b_per_w, NC):
    mesh = plsc.VectorSubcoreMesh(core_axis_name="c", subcore_axis_name="s")
    n_groups = b_per_w // 16
    go_hi = n_groups // 8
    c_per_w = b_per_w // 128

    @functools.partial(
        pl.kernel,
        mesh=mesh,
        out_type=(
            jax.ShapeDtypeStruct((B, DZ + DS), jnp.float32),
            jax.ShapeDtypeStruct((DZ // 8, B // 128, 8, 128), jnp.float32),
            jax.ShapeDtypeStruct((DS // 8, B // 128, 8, 128), jnp.float32),
        ),
        scratch_types=[
            pltpu.VMEM((b_per_w,), jnp.int32),
            pltpu.VMEM((b_per_w,), jnp.int32),
            pltpu.VMEM((V, DZ), jnp.float32),
            pltpu.VMEM((V, DS), jnp.float32),
            pltpu.VMEM((b_per_w, DZ), jnp.float32),
            pltpu.VMEM((b_per_w, DS), jnp.float32),
            pltpu.VMEM((DZ // 8, c_per_w, 8, 128), jnp.float32),
            pltpu.SemaphoreType.DMA,
            pltpu.SemaphoreType.DMA,
            pltpu.SemaphoreType.DMA,
            pltpu.SemaphoreType.DMA,
            pltpu.SemaphoreType.DMA,
        ],
        compiler_params=pltpu.CompilerParams(
            use_tc_tiling_on_sc=False, needs_layout_passes=False),
    )
    def sc_kernel(zi_hbm, li_hbm, zt_hbm, st_hbm, tl_hbm, zo_hbm, so_hbm,
                  zi_v, li_v, zt_v, st_v, z_v, s_v, ot_v,
                  sem_i, sem_t, sem_z, sem_s, sem_o):
        wid = lax.axis_index("s") * NC + lax.axis_index("c")
        base = wid * b_per_w
        ci1 = pltpu.async_copy(zi_hbm.at[pl.ds(base, b_per_w)], zi_v, sem_i)
        ci2 = pltpu.async_copy(li_hbm.at[pl.ds(base, b_per_w)], li_v, sem_i)
        ct1 = pltpu.async_copy(zt_hbm, zt_v, sem_t)
        ct2 = pltpu.async_copy(st_hbm, st_v, sem_t)
        ci1.wait()
        cz = pltpu.async_copy(zt_hbm.at[zi_v], z_v, sem_z)
        ci2.wait()
        cs = pltpu.async_copy(st_hbm.at[li_v], s_v, sem_s)
        cz.wait()
        w1 = pltpu.async_copy(
            z_v, tl_hbm.at[pl.ds(base, b_per_w), pl.ds(0, DZ)], sem_z)
        cs.wait()
        w2 = pltpu.async_copy(
            s_v, tl_hbm.at[pl.ds(base, b_per_w), pl.ds(DZ, DS)], sem_s)

        def assemble(idx_v, tab_v, D):
            def go_body(go, _):
                for gi in range(8):
                    g16 = go * 128 + gi * 16
                    idx = idx_v[pl.ds(g16, 16)]
                    for d in range(D):
                        dsplat = jnp.full((16,), d, jnp.int32)
                        val = plsc.load_gather(tab_v, [idx, dsplat])
                        ot_v[d // 8, go, d % 8, pl.ds(gi * 16, 16)] = val
                return _
            lax.fori_loop(0, go_hi, go_body, 0)

        ct1.wait()
        assemble(zi_v, zt_v, DZ)
        o1 = pltpu.async_copy(ot_v, zo_hbm.at[:, pl.ds(wid * c_per_w, c_per_w)],
                              sem_o)
        o1.wait()
        ct2.wait()
        assemble(li_v, st_v, DS)
        o2 = pltpu.async_copy(ot_v, so_hbm.at[:, pl.ds(wid * c_per_w, c_per_w)],
                              sem_o)
        w1.wait()
        w2.wait()
        o2.wait()

    return sc_kernel


def kernel(sample_indices, batch_size, labels, z_table, s_tissue_table):
    B = sample_indices.shape[0]
    V, DZ = z_table.shape
    DS = s_tissue_table.shape[1]
    info = plsc.get_sparse_core_info()
    NC, NS = info.num_cores, info.num_subcores
    NW = NC * NS
    b_per_w = B // NW

    zi = sample_indices.astype(jnp.int32)
    li = labels[:, 0].astype(jnp.int32)

    sc_kernel = _make_sc_kernel(B, DZ, DS, V, b_per_w, NC)
    total_latent, zo, so = sc_kernel(zi, li, z_table, s_tissue_table)
    z = zo.transpose(1, 3, 0, 2).reshape(B, DZ)
    s = so.transpose(1, 3, 0, 2).reshape(B, DS)
    return (total_latent, z, s)
